# Initial kernel scaffold; baseline (speedup 1.0000x reference)
import jax
import jax.numpy as jnp
from jax.experimental import pallas as pl


def kernel(feat, segment_ids):
    raise NotImplementedError("write your pallas kernel here")


# speedup vs baseline: 11.8441x; 11.8441x over previous
import functools

import jax
import jax.numpy as jnp
from jax import lax
from jax.experimental import pallas as pl
from jax.experimental.pallas import tpu as pltpu
from jax.experimental.pallas import tpu_sc as plsc

_EPS = 1e-06
_N = 50000
_D = 256
_NSEG = 128
_C = 112
_NCHUNK = (_N + _C - 1) // _C
_L = 16
_NW = 32
_JD = _D // _L


def _sc_partial_kernel(feat_hbm, ids_hbm, out_hbm, chunk_v, idx_v, acc_v,
                       semf0, semf1, semi0, semi1):
    cid = lax.axis_index("c")
    sid = lax.axis_index("s")
    wid = sid * 2 + cid
    semf = (semf0, semf1)
    semi = (semi0, semi1)

    zvec = jnp.zeros((_L,), jnp.float32)
    lane = lax.iota(jnp.int32, _L)

    def zero_row(r, carry):
        for j in range(_JD):
            acc_v[pl.ds(r * _D + j * _L, _L)] = zvec
        return carry

    lax.fori_loop(0, 2 * _NSEG, zero_row, 0)

    c_lo = (wid * _NCHUNK) // _NW
    c_hi = ((wid + 1) * _NCHUNK) // _NW

    def start(c, b):
        off = lax.min(c * _C, _N - _C)
        pltpu.async_copy(feat_hbm.at[pl.ds(off, _C)], chunk_v.at[b], semf[b])
        pltpu.async_copy(ids_hbm.at[pl.ds(off, _C)], idx_v.at[b], semi[b])

    def process(c, b):
        off = lax.min(c * _C, _N - _C)
        nzero = c * _C - off
        pltpu.make_async_copy(feat_hbm.at[pl.ds(off, _C)], chunk_v.at[b],
                              semf[b]).wait()
        pltpu.make_async_copy(ids_hbm.at[pl.ds(off, _C)], idx_v.at[b],
                              semi[b]).wait()

        def zrow(r, carry2):
            for j in range(_JD):
                chunk_v[b, r, pl.ds(j * _L, _L)] = zvec
            return carry2

        lax.fori_loop(0, nzero, zrow, 0)

        def group_body(g, carry2):
            seg_vec = idx_v[b, pl.ds(g * _L, _L)]
            seg0 = seg_vec.at[jnp.zeros((_L,), jnp.int32)].get(
                mode="promise_in_bounds")
            uniform = jnp.all(seg_vec == seg0)
            row0 = g * _L

            def fast(_):
                def racc(r, carry):
                    out = []
                    for j in range(_JD):
                        x = chunk_v[b, row0 + r, pl.ds(j * _L, _L)]
                        out.append(carry[j] + x)
                        out.append(carry[_JD + j] + x * x)
                    return tuple(out[0::2]) + tuple(out[1::2])

                acc = lax.fori_loop(0, _L, racc, (zvec,) * (2 * _JD))
                base = seg0 * _D + lane
                for j in range(_JD):
                    plsc.addupdate_scatter(acc_v, [base + (j * _L)], acc[j])
                    plsc.addupdate_scatter(
                        acc_v, [base + (_NSEG * _D + j * _L)], acc[_JD + j])
                return 0

            def slow(_):
                def row_body(r, carry3):
                    ridx = jnp.full((_L,), r, jnp.int32)
                    seg = seg_vec.at[ridx].get(mode="promise_in_bounds")
                    base = seg * _D + lane
                    row = row0 + r
                    for j in range(_JD):
                        x = chunk_v[b, row, pl.ds(j * _L, _L)]
                        plsc.addupdate_scatter(acc_v, [base + (j * _L)], x)
                        plsc.addupdate_scatter(
                            acc_v, [base + (_NSEG * _D + j * _L)], x * x)
                    return carry3

                return lax.fori_loop(0, _L, row_body, 0)

            lax.cond(uniform, fast, slow, 0)
            return carry2

        lax.fori_loop(0, _C // _L, group_body, 0)

    @pl.when(c_lo < c_hi)
    def _():
        start(c_lo, 0)

    def outer(k, carry):
        c = c_lo + 2 * k

        @pl.when(c + 1 < c_hi)
        def _():
            start(c + 1, 1)

        process(c, 0)

        @pl.when(c + 1 < c_hi)
        def _():
            @pl.when(c + 2 < c_hi)
            def _():
                start(c + 2, 0)

            process(c + 1, 1)

        return carry

    npair = (c_hi - c_lo + 1) // 2
    lax.fori_loop(0, npair, outer, 0)

    pltpu.sync_copy(acc_v, out_hbm.at[wid])


_sc_partial = functools.partial(
    pl.kernel,
    out_type=jax.ShapeDtypeStruct((_NW, 2 * _NSEG * _D), jnp.float32),
    mesh=plsc.VectorSubcoreMesh(core_axis_name="c", subcore_axis_name="s"),
    compiler_params=pltpu.CompilerParams(needs_layout_passes=False),
    scratch_types=[
        pltpu.VMEM((2, _C, _D), jnp.float32),
        pltpu.VMEM((2, _C), jnp.int32),
        pltpu.VMEM((2 * _NSEG * _D,), jnp.float32),
        pltpu.SemaphoreType.DMA,
        pltpu.SemaphoreType.DMA,
        pltpu.SemaphoreType.DMA,
        pltpu.SemaphoreType.DMA,
    ],
)(_sc_partial_kernel)


def _epilogue_kernel(p_ref, o_ref):
    acc = jnp.sum(p_ref[...], axis=0)
    s = acc[:_NSEG]
    q = acc[_NSEG:]
    o_ref[...] = jnp.sqrt(jnp.maximum(q - s * s, 0.0) + _EPS)


def kernel(feat, segment_ids):
    ids = segment_ids.astype(jnp.int32)
    partials = _sc_partial(feat, ids).reshape(_NW, 2 * _NSEG, _D)
    return pl.pallas_call(
        _epilogue_kernel,
        out_shape=jax.ShapeDtypeStruct((_NSEG, _D), jnp.float32),
    )(partials)
